# baseline (speedup 1.0000x reference)
import functools

import jax
import jax.numpy as jnp
from jax import lax
from jax.experimental import pallas as pl
from jax.experimental.pallas import tpu as pltpu
from jax.experimental.pallas import tpu_sc as plsc

_E = 320000
_D = 128
_W = _D // 2
_NC = 2
_NS = 16
_L = 16
_NW = _NC * _NS
_EPW = _E // _NW
_C = 80
_CHUNKS = _EPW // _C
_R = 5


def _vsqrt(x):
    xi = lax.bitcast_convert_type(x, jnp.int32)
    yi = (xi >> 1) + jnp.int32(0x1FBD1DF5)
    y = lax.bitcast_convert_type(yi, jnp.float32)
    y = 0.5 * (y + x / y)
    y = 0.5 * (y + x / y)
    return y


_mesh = plsc.VectorSubcoreMesh(core_axis_name="c", subcore_axis_name="s")

_scratch = (
    [
        pltpu.VMEM((_EPW,), jnp.int32),
        pltpu.VMEM((_EPW,), jnp.int32),
    ]
    + [pltpu.VMEM((_C, _W), jnp.int32) for _ in range(_R)]
    + [pltpu.VMEM((_C, _W), jnp.int32) for _ in range(_R)]
    + [
        pltpu.VMEM((_L * _L,), jnp.float32),
        pltpu.VMEM((_L,), jnp.float32),
    ]
    + [pltpu.SemaphoreType.DMA for _ in range(_R)]
    + [pltpu.SemaphoreType.DMA]
)


@functools.partial(
    pl.kernel,
    out_type=jax.ShapeDtypeStruct((_NW, _L), jnp.float32),
    mesh=_mesh,
    compiler_params=pltpu.CompilerParams(
        needs_layout_passes=False, use_tc_tiling_on_sc=False),
    scratch_types=_scratch,
)
def _sc_loss(feat_hbm, eidx_hbm, out_hbm, *scr):
    sidx_all, didx_all = scr[0], scr[1]
    sbuf = scr[2:2 + _R]
    dbuf = scr[2 + _R:2 + 2 * _R]
    tmp = scr[2 + 2 * _R]
    tot_v = scr[3 + 2 * _R]
    semG = scr[4 + 2 * _R:4 + 3 * _R]
    semI = scr[4 + 3 * _R]

    wid = lax.axis_index("s") * _NC + lax.axis_index("c")
    wbase = wid * _EPW
    lane = lax.iota(jnp.int32, _L)

    def issue_gathers(n, k):
        pltpu.async_copy(
            feat_hbm.at[sidx_all.at[pl.ds(n * _C, _C)]], sbuf[k], semG[k])
        pltpu.async_copy(
            feat_hbm.at[didx_all.at[pl.ds(n * _C, _C)]], dbuf[k], semG[k])

    def wait_gathers(k):
        pltpu.make_async_copy(
            feat_hbm.at[sidx_all.at[pl.ds(0, _C)]], sbuf[k], semG[k]).wait()
        pltpu.make_async_copy(
            feat_hbm.at[didx_all.at[pl.ds(0, _C)]], dbuf[k], semG[k]).wait()

    def compute(k, total):
        srows = sbuf[k]
        drows = dbuf[k]

        def group_body(i, tot):
            base = i * _L
            for j in range(_L):
                row = base + j
                acc0 = acc1 = None
                for b in range(_W // _L):
                    s = plsc.bitcast(srows[row, pl.ds(b * _L, _L)],
                                     jnp.bfloat16)
                    t = plsc.bitcast(drows[row, pl.ds(b * _L, _L)],
                                     jnp.bfloat16)
                    df = s - t
                    sq = df * df
                    lo, hi = plsc.unpack(sq,
                                         format=plsc.PackFormat.INTERLEAVED)
                    acc0 = lo if acc0 is None else acc0 + lo
                    acc1 = hi if acc1 is None else acc1 + hi
                plsc.store_scatter(tmp, [lane * _L + j], acc0 + acc1)
            sq16 = tmp[pl.ds(0, _L)]
            for l in range(1, _L):
                sq16 = sq16 + tmp[pl.ds(l * _L, _L)]
            return tot + _vsqrt(sq16)

        return lax.fori_loop(0, _C // _L, group_body, total)

    pltpu.async_copy(eidx_hbm.at[pl.ds(wbase, _EPW)], sidx_all, semI)
    pltpu.async_copy(eidx_hbm.at[pl.ds(_E + wbase, _EPW)], didx_all, semI)
    pltpu.make_async_copy(eidx_hbm.at[pl.ds(0, _EPW)], sidx_all, semI).wait()
    pltpu.make_async_copy(eidx_hbm.at[pl.ds(0, _EPW)], didx_all, semI).wait()
    for i in range(4):
        issue_gathers(i, i)

    def ring_body(p, total):
        n0 = p * _R
        for k in range(_R):
            n = n0 + k

            @pl.when(n + 4 < _CHUNKS)
            def _():
                issue_gathers(n + 4, (k + 4) % _R)

            wait_gathers(k)
            total = compute(k, total)
        return total

    total = lax.fori_loop(0, _CHUNKS // _R, ring_body,
                          jnp.zeros((_L,), jnp.float32))

    tot_v[...] = total
    pltpu.sync_copy(tot_v, out_hbm.at[wid])


def kernel(features, edge_index):
    fb = features.astype(jnp.bfloat16).reshape(features.shape[0], _W, 2)
    fw = lax.bitcast_convert_type(fb, jnp.int32)
    partials = _sc_loss(fw, edge_index.reshape(-1))
    return jnp.sum(partials) * (1.0 / _E)


# speedup vs baseline: 1.0769x; 1.0769x over previous
import functools

import jax
import jax.numpy as jnp
from jax import lax
from jax.experimental import pallas as pl
from jax.experimental.pallas import tpu as pltpu
from jax.experimental.pallas import tpu_sc as plsc

_E = 320000
_D = 128
_NC = 2
_NS = 16
_L = 16
_NW = _NC * _NS
_EPW = _E // _NW
_C = 80
_CHUNKS = _EPW // _C
_R = 5


def _vsqrt(x):
    xi = lax.bitcast_convert_type(x, jnp.int32)
    yi = (xi >> 1) + jnp.int32(0x1FBD1DF5)
    y = lax.bitcast_convert_type(yi, jnp.float32)
    y = 0.5 * (y + x / y)
    y = 0.5 * (y + x / y)
    return y


_mesh = plsc.VectorSubcoreMesh(core_axis_name="c", subcore_axis_name="s")

_scratch = (
    [
        pltpu.VMEM((_EPW,), jnp.int32),
        pltpu.VMEM((_EPW,), jnp.int32),
    ]
    + [pltpu.VMEM((_C, _D), jnp.float32) for _ in range(_R)]
    + [
        pltpu.VMEM((_L * _L,), jnp.float32),
        pltpu.VMEM((_L,), jnp.float32),
    ]
    + [pltpu.SemaphoreType.DMA for _ in range(_R)]
    + [pltpu.SemaphoreType.DMA]
)


@functools.partial(
    pl.kernel,
    out_type=jax.ShapeDtypeStruct((_NW, _L), jnp.float32),
    mesh=_mesh,
    compiler_params=pltpu.CompilerParams(needs_layout_passes=False),
    scratch_types=_scratch,
)
def _sc_loss(feat_hbm, fneg_hbm, eidx_hbm, out_hbm, *scr):
    sidx_all, didx_all = scr[0], scr[1]
    dbuf = scr[2:2 + _R]
    tmp = scr[2 + _R]
    tot_v = scr[3 + _R]
    semG = scr[4 + _R:4 + 2 * _R]
    semI = scr[4 + 2 * _R]

    wid = lax.axis_index("s") * _NC + lax.axis_index("c")
    wbase = wid * _EPW
    lane = lax.iota(jnp.int32, _L)

    def issue_g1(n, k):
        pltpu.async_copy(
            feat_hbm.at[sidx_all.at[pl.ds(n * _C, _C)]], dbuf[k], semG[k])

    def wait_g1(k):
        pltpu.make_async_copy(
            feat_hbm.at[sidx_all.at[pl.ds(0, _C)]], dbuf[k], semG[k]).wait()

    def issue_g2(n, k):
        pltpu.async_copy(
            fneg_hbm.at[didx_all.at[pl.ds(n * _C, _C)]], dbuf[k], semG[k],
            add=True)

    def wait_g2(k):
        pltpu.make_async_copy(
            fneg_hbm.at[didx_all.at[pl.ds(0, _C)]], dbuf[k], semG[k]).wait()

    def compute(k, total):
        rows = dbuf[k]

        def group_body(i, tot):
            base = i * _L
            for j in range(_L):
                row = base + j
                acc = None
                for b in range(_D // _L):
                    df = rows[row, pl.ds(b * _L, _L)]
                    sq = df * df
                    acc = sq if acc is None else acc + sq
                plsc.store_scatter(tmp, [lane * _L + j], acc)
            sq16 = tmp[pl.ds(0, _L)]
            for l in range(1, _L):
                sq16 = sq16 + tmp[pl.ds(l * _L, _L)]
            return tot + _vsqrt(sq16)

        return lax.fori_loop(0, _C // _L, group_body, total)

    pltpu.async_copy(eidx_hbm.at[pl.ds(wbase, _EPW)], sidx_all, semI)
    pltpu.async_copy(eidx_hbm.at[pl.ds(_E + wbase, _EPW)], didx_all, semI)
    pltpu.make_async_copy(eidx_hbm.at[pl.ds(0, _EPW)], sidx_all, semI).wait()
    pltpu.make_async_copy(eidx_hbm.at[pl.ds(0, _EPW)], didx_all, semI).wait()
    for i in range(4):
        issue_g1(i, i)
    for i in range(3):
        wait_g1(i)
        issue_g2(i, i)

    def ring_body(p, total):
        n0 = p * _R
        for k in range(_R):
            n = n0 + k

            @pl.when(n + 4 < _CHUNKS)
            def _():
                issue_g1(n + 4, (k + 4) % _R)

            @pl.when(n + 3 < _CHUNKS)
            def _():
                wait_g1((k + 3) % _R)
                issue_g2(n + 3, (k + 3) % _R)

            wait_g2(k)
            total = compute(k, total)
        return total

    total = lax.fori_loop(0, _CHUNKS // _R, ring_body,
                          jnp.zeros((_L,), jnp.float32))

    tot_v[...] = total
    pltpu.sync_copy(tot_v, out_hbm.at[wid])


def kernel(features, edge_index):
    partials = _sc_loss(features, -features, edge_index.reshape(-1))
    return jnp.sum(partials) * (1.0 / _E)


# speedup vs baseline: 1.1520x; 1.0697x over previous
import functools

import jax
import jax.numpy as jnp
from jax import lax
from jax.experimental import pallas as pl
from jax.experimental.pallas import tpu as pltpu
from jax.experimental.pallas import tpu_sc as plsc

_E = 320000
_D = 128
_NC = 2
_NS = 16
_L = 16
_NW = _NC * _NS
_EPW = _E // _NW
_C = 80
_CHUNKS = _EPW // _C
_R = 5


def _vsqrt(x):
    xi = lax.bitcast_convert_type(x, jnp.int32)
    yi = (xi >> 1) + jnp.int32(0x1FBD1DF5)
    y = lax.bitcast_convert_type(yi, jnp.float32)
    y = 0.5 * (y + x / y)
    y = 0.5 * (y + x / y)
    return y


_mesh = plsc.VectorSubcoreMesh(core_axis_name="c", subcore_axis_name="s")

_scratch = (
    [
        pltpu.VMEM((_EPW,), jnp.int32),
        pltpu.VMEM((_EPW,), jnp.int32),
    ]
    + [pltpu.VMEM((_C, _D), jnp.float32) for _ in range(_R)]
    + [
        pltpu.VMEM((_L * _L,), jnp.float32),
        pltpu.VMEM((_L,), jnp.float32),
    ]
    + [pltpu.SemaphoreType.DMA for _ in range(_R)]
    + [pltpu.SemaphoreType.DMA]
)


@functools.partial(
    pl.kernel,
    out_type=jax.ShapeDtypeStruct((_NW, _L), jnp.float32),
    mesh=_mesh,
    compiler_params=pltpu.CompilerParams(needs_layout_passes=False),
    scratch_types=_scratch,
)
def _sc_loss(feat_hbm, fneg_hbm, eidx_hbm, out_hbm, *scr):
    sidx_all, didx_all = scr[0], scr[1]
    dbuf = scr[2:2 + _R]
    tmp = scr[2 + _R]
    tot_v = scr[3 + _R]
    semG = scr[4 + _R:4 + 2 * _R]
    semI = scr[4 + 2 * _R]

    wid = lax.axis_index("s") * _NC + lax.axis_index("c")
    wbase = wid * _EPW
    lane = lax.iota(jnp.int32, _L)

    def issue_g1(n, k):
        pltpu.async_copy(
            feat_hbm.at[sidx_all.at[pl.ds(n * _C, _C)]], dbuf[k], semG[k])

    def wait_g1(k):
        pltpu.make_async_copy(
            feat_hbm.at[sidx_all.at[pl.ds(0, _C)]], dbuf[k], semG[k]).wait()

    def issue_g2(n, k):
        pltpu.async_copy(
            fneg_hbm.at[didx_all.at[pl.ds(n * _C, _C)]], dbuf[k], semG[k],
            add=True)

    def wait_g2(k):
        pltpu.make_async_copy(
            fneg_hbm.at[didx_all.at[pl.ds(0, _C)]], dbuf[k], semG[k]).wait()

    def compute(k, total):
        rows = dbuf[k]

        def group_body(i, tot):
            base = i * _L
            for j in range(_L):
                row = base + j
                acc = None
                for b in range(_D // _L):
                    df = rows[row, pl.ds(b * _L, _L)]
                    sq = df * df
                    acc = sq if acc is None else acc + sq
                plsc.store_scatter(tmp, [lane * _L + j], acc)
            sq16 = tmp[pl.ds(0, _L)]
            for l in range(1, _L):
                sq16 = sq16 + tmp[pl.ds(l * _L, _L)]
            return tot + _vsqrt(sq16)

        return lax.fori_loop(0, _C // _L, group_body, total)

    pltpu.async_copy(eidx_hbm.at[pl.ds(wbase, _EPW)], sidx_all, semI)
    pltpu.async_copy(eidx_hbm.at[pl.ds(_E + wbase, _EPW)], didx_all, semI)
    pltpu.make_async_copy(eidx_hbm.at[pl.ds(0, _EPW)], sidx_all, semI).wait()
    pltpu.make_async_copy(eidx_hbm.at[pl.ds(0, _EPW)], didx_all, semI).wait()
    for i in range(4):
        issue_g1(i, i)
    for i in range(2):
        wait_g1(i)
        issue_g2(i, i)

    def ring_body(p, total):
        n0 = p * _R
        for k in range(_R):
            n = n0 + k

            @pl.when(n + 2 < _CHUNKS)
            def _():
                wait_g1((k + 2) % _R)
                issue_g2(n + 2, (k + 2) % _R)

            @pl.when(n + 4 < _CHUNKS)
            def _():
                issue_g1(n + 4, (k + 4) % _R)

            wait_g2(k)
            total = compute(k, total)
        return total

    total = lax.fori_loop(0, _CHUNKS // _R, ring_body,
                          jnp.zeros((_L,), jnp.float32))

    tot_v[...] = total
    pltpu.sync_copy(tot_v, out_hbm.at[wid])


def kernel(features, edge_index):
    partials = _sc_loss(features, -features, edge_index.reshape(-1))
    return jnp.sum(partials) * (1.0 / _E)
